# baseline (speedup 1.0000x reference)
import functools

import jax
import jax.numpy as jnp
from jax import lax
from jax.experimental import pallas as pl
from jax.experimental.pallas import tpu as pltpu
from jax.experimental.pallas import tpu_sc as plsc

C = 96
K = 17
B = 4
HW = 224 * 224
NUM_WORKERS = 32
CH_PER_W = C // NUM_WORKERS
NBUF = 4
CHUNK = HW // 2
CHUNKS_PER_W = CH_PER_W * 2
CVECS = CHUNK // 16


def _sc_body(x_hbm, t_hbm, out_hbm, b0, b1, b2, b3, tbl,
             s0, s1, s2, s3, so0, so1, so2, so3):
    nc = 2
    wid = lax.axis_index("s") * nc + lax.axis_index("c")
    c0 = wid * CH_PER_W

    bufs = (b0, b1, b2, b3)
    isems = (s0, s1, s2, s3)
    osems = (so0, so1, so2, so3)

    def src_of(t):
        return (c0 + t // 2) * HW + (t % 2) * CHUNK

    def in_copy(t, buf, sem):
        return pltpu.make_async_copy(x_hbm.at[pl.ds(src_of(t), CHUNK)], buf, sem)

    def out_copy(t, buf, sem):
        return pltpu.make_async_copy(buf, out_hbm.at[pl.ds(src_of(t), CHUNK)], sem)

    pltpu.sync_copy(t_hbm.at[pl.ds(c0 * 1024, 3 * 1024)], tbl)

    lane = jax.lax.iota(jnp.int32, 16)

    def compute_chunk(buf, k):
        base = lane + k * 1024
        bv7 = plsc.load_gather(tbl, [base + 7 * 16])
        bv3 = plsc.load_gather(tbl, [base + 3 * 16])
        bv11 = plsc.load_gather(tbl, [base + 11 * 16])
        base128 = base + 8 * 16

        @plsc.parallel_loop(0, CVECS, step=1, unroll=8)
        def cbody(i):
            off = i * 16
            vv = buf[pl.ds(off, 16)]
            m8 = bv7 <= vv
            g = jnp.where(m8, base128, base)
            t4 = jnp.where(m8, bv11, bv3)
            m4 = t4 <= vv
            g = jnp.where(m4, g + 4 * 16, g)
            t2 = plsc.load_gather(tbl, [g + 16])
            g = jnp.where(t2 <= vv, g + 2 * 16, g)
            t1 = plsc.load_gather(tbl, [g])
            g = jnp.where(t1 <= vv, g + 16, g)
            sl = plsc.load_gather(tbl, [g + 256])
            sxv = plsc.load_gather(tbl, [g + 512])
            syv = plsc.load_gather(tbl, [g + 768])
            buf[pl.ds(off, 16)] = syv + sl * (vv - sxv)

    in_copy(0, bufs[0], isems[0]).start()
    in_copy(1, bufs[1], isems[1]).start()
    in_copy(2, bufs[2], isems[2]).start()
    for t in range(CHUNKS_PER_W):
        p = t % NBUF
        in_copy(t, bufs[p], isems[p]).wait()
        compute_chunk(bufs[p], t // 2)
        out_copy(t, bufs[p], osems[p]).start()
        if t + 3 < CHUNKS_PER_W:
            q = (t + 3) % NBUF
            if t >= 1:
                out_copy(t - 1, bufs[q], osems[q]).wait()
            in_copy(t + 3, bufs[q], isems[q]).start()
    for t in range(CHUNKS_PER_W - 3, CHUNKS_PER_W):
        p = t % NBUF
        out_copy(t, bufs[p], osems[p]).wait()


@functools.cache
def _build_sc_call():
    mesh = plsc.VectorSubcoreMesh(core_axis_name="c", subcore_axis_name="s")
    return pl.kernel(
        _sc_body,
        out_type=jax.ShapeDtypeStruct((C * HW,), jnp.float32),
        mesh=mesh,
        compiler_params=pltpu.CompilerParams(needs_layout_passes=False),
        scratch_types=[
            pltpu.VMEM((CHUNK,), jnp.float32),
            pltpu.VMEM((CHUNK,), jnp.float32),
            pltpu.VMEM((CHUNK,), jnp.float32),
            pltpu.VMEM((CHUNK,), jnp.float32),
            pltpu.VMEM((3 * 1024,), jnp.float32),
            pltpu.SemaphoreType.DMA,
            pltpu.SemaphoreType.DMA,
            pltpu.SemaphoreType.DMA,
            pltpu.SemaphoreType.DMA,
            pltpu.SemaphoreType.DMA,
            pltpu.SemaphoreType.DMA,
            pltpu.SemaphoreType.DMA,
            pltpu.SemaphoreType.DMA,
        ],
    )


def kernel(x, peer_x, peer_y):
    xp = jnp.sort(peer_y, axis=1)
    yp = jnp.sort(peer_x, axis=1)
    sx = xp[:, : K - 1]
    ex = xp[:, 1:]
    sy = yp[:, : K - 1]
    ey = yp[:, 1:]
    slope = (ey - sy) / (ex - sx)
    bp = jnp.concatenate(
        [xp[:, 1 : K - 1], jnp.full((C, 1), jnp.inf, jnp.float32)], axis=1
    )
    tables = jnp.stack([bp, slope, sx, sy], axis=1)
    tables = jnp.broadcast_to(tables[..., None], (C, 4, 16, 16)).reshape(C * 1024)

    call = _build_sc_call()
    flats = []
    xg = x
    for b in range(B):
        f = xg[b].reshape(C * HW)
        flats.append(f)
        if b + 1 < B:
            xg, _ = lax.optimization_barrier((xg, f))
    out = jnp.zeros((B, C, 224, 224), jnp.float32)
    for b in range(B):
        yb = call(flats[b], tables).reshape(1, C, 224, 224)
        out = lax.dynamic_update_slice(out, yb, (b, 0, 0, 0))
    return out


# speedup vs baseline: 1.0933x; 1.0933x over previous
import functools

import jax
import jax.numpy as jnp
from jax import lax
from jax.experimental import pallas as pl
from jax.experimental.pallas import tpu as pltpu
from jax.experimental.pallas import tpu_sc as plsc

C = 96
K = 17
B = 4
HW = 224 * 224
NUM_WORKERS = 32
CH_PER_W = C // NUM_WORKERS
NBUF = 4
CHUNK = HW // 2
CHUNKS_PER_W = CH_PER_W * 2
CVECS = CHUNK // 16


def _sc_body(x_hbm, t_hbm, out_hbm, b0, b1, b2, b3, tbl,
             s0, s1, s2, s3, so0, so1, so2, so3):
    nc = 2
    wid = lax.axis_index("s") * nc + lax.axis_index("c")
    c0 = wid * CH_PER_W

    bufs = (b0, b1, b2, b3)
    isems = (s0, s1, s2, s3)
    osems = (so0, so1, so2, so3)

    def src_of(t):
        return (c0 + t // 2) * HW + (t % 2) * CHUNK

    def in_copy(t, buf, sem):
        return pltpu.make_async_copy(x_hbm.at[pl.ds(src_of(t), CHUNK)], buf, sem)

    def out_copy(t, buf, sem):
        return pltpu.make_async_copy(buf, out_hbm.at[pl.ds(src_of(t), CHUNK)], sem)

    pltpu.sync_copy(t_hbm.at[pl.ds(c0 * 1024, 3 * 1024)], tbl)

    lane = jax.lax.iota(jnp.int32, 16)

    def compute_chunk(buf, k):
        base = lane + k * 1024
        bv7 = plsc.load_gather(tbl, [base + 7 * 16])
        bv3 = plsc.load_gather(tbl, [base + 3 * 16])
        bv11 = plsc.load_gather(tbl, [base + 11 * 16])
        base128 = base + 8 * 16

        @plsc.parallel_loop(0, CVECS, step=1, unroll=8)
        def cbody(i):
            off = i * 16
            vv = buf[pl.ds(off, 16)]
            m8 = bv7 <= vv
            g = jnp.where(m8, base128, base)
            t4 = jnp.where(m8, bv11, bv3)
            m4 = t4 <= vv
            g = jnp.where(m4, g + 4 * 16, g)
            t2 = plsc.load_gather(tbl, [g + 16])
            g = jnp.where(t2 <= vv, g + 2 * 16, g)
            t1 = plsc.load_gather(tbl, [g])
            g = jnp.where(t1 <= vv, g + 16, g)
            sl = plsc.load_gather(tbl, [g + 256])
            sxv = plsc.load_gather(tbl, [g + 512])
            syv = plsc.load_gather(tbl, [g + 768])
            buf[pl.ds(off, 16)] = syv + sl * (vv - sxv)

    in_copy(0, bufs[0], isems[0]).start()
    in_copy(1, bufs[1], isems[1]).start()
    in_copy(2, bufs[2], isems[2]).start()
    for t in range(CHUNKS_PER_W):
        p = t % NBUF
        in_copy(t, bufs[p], isems[p]).wait()
        compute_chunk(bufs[p], t // 2)
        out_copy(t, bufs[p], osems[p]).start()
        if t + 3 < CHUNKS_PER_W:
            q = (t + 3) % NBUF
            if t >= 1:
                out_copy(t - 1, bufs[q], osems[q]).wait()
            in_copy(t + 3, bufs[q], isems[q]).start()
    for t in range(CHUNKS_PER_W - 3, CHUNKS_PER_W):
        p = t % NBUF
        out_copy(t, bufs[p], osems[p]).wait()


@functools.cache
def _build_sc_call():
    mesh = plsc.VectorSubcoreMesh(core_axis_name="c", subcore_axis_name="s")
    return pl.kernel(
        _sc_body,
        out_type=jax.ShapeDtypeStruct((C * HW,), jnp.float32),
        mesh=mesh,
        compiler_params=pltpu.CompilerParams(needs_layout_passes=False),
        scratch_types=[
            pltpu.VMEM((CHUNK,), jnp.float32),
            pltpu.VMEM((CHUNK,), jnp.float32),
            pltpu.VMEM((CHUNK,), jnp.float32),
            pltpu.VMEM((CHUNK,), jnp.float32),
            pltpu.VMEM((3 * 1024,), jnp.float32),
            pltpu.SemaphoreType.DMA,
            pltpu.SemaphoreType.DMA,
            pltpu.SemaphoreType.DMA,
            pltpu.SemaphoreType.DMA,
            pltpu.SemaphoreType.DMA,
            pltpu.SemaphoreType.DMA,
            pltpu.SemaphoreType.DMA,
            pltpu.SemaphoreType.DMA,
        ],
    )


def kernel(x, peer_x, peer_y):
    xp = jnp.sort(peer_y, axis=1)
    yp = jnp.sort(peer_x, axis=1)
    sx = xp[:, : K - 1]
    ex = xp[:, 1:]
    sy = yp[:, : K - 1]
    ey = yp[:, 1:]
    slope = (ey - sy) / (ex - sx)
    bp = jnp.concatenate(
        [xp[:, 1 : K - 1], jnp.full((C, 1), jnp.inf, jnp.float32)], axis=1
    )
    tables = jnp.stack([bp, slope, sx, sy], axis=1)
    tables = jnp.broadcast_to(tables[..., None], (C, 4, 16, 16)).reshape(C * 1024)

    call = _build_sc_call()
    flats = []
    xg = x
    for b in range(B):
        f = xg[b].reshape(C * HW)
        flats.append(f)
        if b + 1 < B:
            xg, _ = lax.optimization_barrier((xg, f))
    outs = [call(f, tables).reshape(C, 224, 224) for f in flats]
    return jnp.stack(outs, axis=0)
